# baseline (speedup 1.0000x reference)
import functools

import jax
import jax.numpy as jnp
from jax import lax
from jax.experimental import pallas as pl
from jax.experimental.pallas import tpu as pltpu
from jax.experimental.pallas import tpu_sc as plsc

D = 32
BATCH_PER_CHUNK = 16
L = 16


@functools.cache
def _build(NB, S, V):
    info = plsc.get_sparse_core_info()
    NC, NS = info.num_cores, info.num_subcores
    NW = NC * NS
    assert NB % (NW * BATCH_PER_CHUNK) == 0
    nb_per_w = NB // NW
    n_chunks = nb_per_w // BATCH_PER_CHUNK
    CHUNK = BATCH_PER_CHUNK * S
    assert CHUNK % L == 0
    n_groups = CHUNK // L

    mesh = plsc.VectorSubcoreMesh(core_axis_name="c", subcore_axis_name="s")

    @functools.partial(
        pl.kernel,
        mesh=mesh,
        out_type=jax.ShapeDtypeStruct((NB, S, D), jnp.float32),
        scratch_types=[
            pltpu.VMEM((CHUNK,), jnp.int32),
            pltpu.VMEM((CHUNK,), jnp.int32),
            pltpu.VMEM((CHUNK, 128), jnp.float32),
            pltpu.VMEM((CHUNK, D), jnp.float32),
            pltpu.SemaphoreType.DMA,
            pltpu.SemaphoreType.DMA,
        ],
        compiler_params=pltpu.CompilerParams(
            use_tc_tiling_on_sc=True, needs_layout_passes=False),
    )
    def gather_kernel(hi_hbm, q32_hbm, table_hbm, out_hbm,
                      hi_v, q32_v, r128_v, out32_v, sem_g, sem_s):
        wid = lax.axis_index("s") * NC + lax.axis_index("c")
        batch0 = wid * nb_per_w

        def chunk_body(i, carry):
            b0 = batch0 + i * BATCH_PER_CHUNK
            off = b0 * S
            pltpu.sync_copy(hi_hbm.at[pl.ds(off, CHUNK)], hi_v)
            pltpu.sync_copy(q32_hbm.at[pl.ds(off, CHUNK)], q32_v)
            pltpu.async_copy(table_hbm.at[hi_v], r128_v, sem_g).wait()

            def group_body(g, carry2):
                row_v = lax.iota(jnp.int32, L) + g * L
                q32 = q32_v[pl.ds(g * L, L)]
                for c in range(D):
                    v = plsc.load_gather(r128_v, [row_v, q32 + c])
                    plsc.store_scatter(
                        out32_v, [row_v, jnp.full((L,), c, jnp.int32)], v)
                return carry2

            lax.fori_loop(0, n_groups, group_body, 0)

            descs = []
            for k in range(BATCH_PER_CHUNK):
                descs.append(pltpu.async_copy(
                    out32_v.at[pl.ds(k * S, S), :], out_hbm.at[b0 + k], sem_s))
            for d in descs:
                d.wait()
            return carry

        lax.fori_loop(0, n_chunks, chunk_body, 0)

    return gather_kernel


def kernel(x, embeddings):
    nb, s = x.shape
    V, d = embeddings.shape
    xf = x.reshape(nb * s).astype(jnp.int32)
    hi = xf >> 2
    q32 = (xf & 3) * D
    table128 = embeddings.reshape(V * d // 128, 128)
    return _build(nb, s, V)(hi, q32, table128)


# speedup vs baseline: 1.1253x; 1.1253x over previous
import functools

import jax
import jax.numpy as jnp
from jax import lax
from jax.experimental import pallas as pl
from jax.experimental.pallas import tpu as pltpu
from jax.experimental.pallas import tpu_sc as plsc

D = 32
BATCH_PER_CHUNK = 8
L = 16


@functools.cache
def _build(NB, S, V):
    info = plsc.get_sparse_core_info()
    NC, NS = info.num_cores, info.num_subcores
    NW = NC * NS
    assert NB % (NW * BATCH_PER_CHUNK * 2) == 0
    nb_per_w = NB // NW
    n_chunks = nb_per_w // BATCH_PER_CHUNK
    CHUNK = BATCH_PER_CHUNK * S
    assert CHUNK % L == 0
    n_groups = CHUNK // L

    mesh = plsc.VectorSubcoreMesh(core_axis_name="c", subcore_axis_name="s")

    @functools.partial(
        pl.kernel,
        mesh=mesh,
        out_type=jax.ShapeDtypeStruct((NB, S, D), jnp.float32),
        scratch_types=[
            pltpu.VMEM((CHUNK,), jnp.int32),
            pltpu.VMEM((CHUNK,), jnp.int32),
            pltpu.VMEM((CHUNK,), jnp.int32),
            pltpu.VMEM((CHUNK,), jnp.int32),
            pltpu.VMEM((CHUNK, 128), jnp.float32),
            pltpu.VMEM((CHUNK, 128), jnp.float32),
            pltpu.VMEM((CHUNK, D), jnp.float32),
            pltpu.VMEM((CHUNK, D), jnp.float32),
            pltpu.SemaphoreType.DMA,
            pltpu.SemaphoreType.DMA,
            pltpu.SemaphoreType.DMA,
            pltpu.SemaphoreType.DMA,
            pltpu.SemaphoreType.DMA,
            pltpu.SemaphoreType.DMA,
        ],
        compiler_params=pltpu.CompilerParams(
            use_tc_tiling_on_sc=True, needs_layout_passes=False),
    )
    def gather_kernel(hi_hbm, q32_hbm, table_hbm, out_hbm,
                      hi0, hi1, q0, q1, r0, r1, o0, o1,
                      si0, si1, sg0, sg1, ss0, ss1):
        hi_v, q32_v = [hi0, hi1], [q0, q1]
        r128_v, out32_v = [r0, r1], [o0, o1]
        sem_i, sem_g, sem_s = [si0, si1], [sg0, sg1], [ss0, ss1]
        wid = lax.axis_index("s") * NC + lax.axis_index("c")
        batch0 = wid * nb_per_w

        def idx_start(i, slot):
            off = (batch0 + i * BATCH_PER_CHUNK) * S
            pltpu.async_copy(hi_hbm.at[pl.ds(off, CHUNK)], hi_v[slot],
                             sem_i[slot])
            pltpu.async_copy(q32_hbm.at[pl.ds(off, CHUNK)], q32_v[slot],
                             sem_i[slot])

        def idx_wait(slot):
            pltpu.make_async_copy(hi_hbm.at[pl.ds(0, CHUNK)], hi_v[slot],
                                  sem_i[slot]).wait()
            pltpu.make_async_copy(q32_hbm.at[pl.ds(0, CHUNK)], q32_v[slot],
                                  sem_i[slot]).wait()

        def gather_start(slot):
            pltpu.async_copy(table_hbm.at[hi_v[slot]], r128_v[slot],
                             sem_g[slot])

        def gather_wait(slot):
            pltpu.make_async_copy(table_hbm.at[hi_v[slot]], r128_v[slot],
                                  sem_g[slot]).wait()

        def extract(slot):
            def group_body(g, carry):
                row_v = lax.iota(jnp.int32, L) + g * L
                q32 = q32_v[slot][pl.ds(g * L, L)]
                for c in range(D):
                    v = plsc.load_gather(r128_v[slot], [row_v, q32 + c])
                    plsc.store_scatter(
                        out32_v[slot], [row_v, jnp.full((L,), c, jnp.int32)],
                        v)
                return carry
            lax.fori_loop(0, n_groups, group_body, 0)

        def out_start(i, slot):
            b0 = batch0 + i * BATCH_PER_CHUNK
            for k in range(BATCH_PER_CHUNK):
                pltpu.async_copy(out32_v[slot].at[pl.ds(k * S, S), :],
                                 out_hbm.at[b0 + k], sem_s[slot])

        def out_wait(slot):
            for k in range(BATCH_PER_CHUNK):
                pltpu.make_async_copy(out32_v[slot].at[pl.ds(k * S, S), :],
                                      out_hbm.at[batch0 + k],
                                      sem_s[slot]).wait()

        idx_start(0, 0)
        idx_start(1, 1)
        idx_wait(0)
        gather_start(0)

        def chunk_step(i, slot):
            gather_wait(slot)

            @pl.when(i + 1 < n_chunks)
            def _():
                idx_wait(slot ^ 1)
                gather_start(slot ^ 1)

            @pl.when(i >= 2)
            def _():
                out_wait(slot)

            extract(slot)

            @pl.when(i + 2 < n_chunks)
            def _():
                idx_start(i + 2, slot)

            out_start(i, slot)

        def pair_body(j, carry):
            chunk_step(2 * j, 0)
            chunk_step(2 * j + 1, 1)
            return carry

        lax.fori_loop(0, n_chunks // 2, pair_body, 0)
        out_wait(0)
        out_wait(1)

    return gather_kernel


def kernel(x, embeddings):
    nb, s = x.shape
    V, d = embeddings.shape
    xf = x.reshape(nb * s).astype(jnp.int32)
    hi = xf >> 2
    q32 = (xf & 3) * D
    table128 = embeddings.reshape(V * d // 128, 128)
    return _build(nb, s, V)(hi, q32, table128)


# speedup vs baseline: 1.7360x; 1.5428x over previous
import functools

import jax
import jax.numpy as jnp
from jax import lax
from jax.experimental import pallas as pl
from jax.experimental.pallas import tpu as pltpu
from jax.experimental.pallas import tpu_sc as plsc

D = 32
BATCH_PER_CHUNK = 64


@functools.cache
def _build(NB, S, V):
    info = plsc.get_sparse_core_info()
    NC, NS = info.num_cores, info.num_subcores
    NW = NC * NS
    nb_per_w = NB // NW
    assert nb_per_w % BATCH_PER_CHUNK == 0
    n_chunks = nb_per_w // BATCH_PER_CHUNK
    CHUNK = BATCH_PER_CHUNK * S

    mesh = plsc.VectorSubcoreMesh(core_axis_name="c", subcore_axis_name="s")

    @functools.partial(
        pl.kernel,
        mesh=mesh,
        out_type=jax.ShapeDtypeStruct((NB, S, D), jnp.float32),
        scratch_types=[
            pltpu.VMEM((CHUNK,), jnp.int32),
            pltpu.VMEM((CHUNK,), jnp.int32),
            pltpu.VMEM((CHUNK, D), jnp.float32),
            pltpu.VMEM((CHUNK, D), jnp.float32),
            pltpu.SemaphoreType.DMA,
            pltpu.SemaphoreType.DMA,
            pltpu.SemaphoreType.DMA,
            pltpu.SemaphoreType.DMA,
            pltpu.SemaphoreType.DMA,
            pltpu.SemaphoreType.DMA,
        ],
        compiler_params=pltpu.CompilerParams(use_tc_tiling_on_sc=False),
    )
    def gather_kernel(idx_hbm, table_hbm, out_hbm,
                      i0, i1, r0, r1, si0, si1, sg0, sg1, ss0, ss1):
        idx_bufs, rows_bufs = [i0, i1], [r0, r1]
        sem_i, sem_g, sem_s = [si0, si1], [sg0, sg1], [ss0, ss1]
        wid = lax.axis_index("s") * NC + lax.axis_index("c")
        base = wid * nb_per_w

        def idx_start(i):
            off = (base + i * BATCH_PER_CHUNK) * S
            return pltpu.async_copy(
                idx_hbm.at[pl.ds(off, CHUNK)], idx_bufs[i % 2], sem_i[i % 2])

        def gather_start(i):
            return pltpu.async_copy(
                table_hbm.at[idx_bufs[i % 2]], rows_bufs[i % 2], sem_g[i % 2])

        def store_start(i):
            b0 = base + i * BATCH_PER_CHUNK
            descs = []
            for k in range(BATCH_PER_CHUNK):
                descs.append(pltpu.async_copy(
                    rows_bufs[i % 2].at[pl.ds(k * S, S), :],
                    out_hbm.at[b0 + k], sem_s[i % 2]))
            return descs

        d_idx, d_g, d_s = {}, {}, {}
        d_idx[0] = idx_start(0)
        if n_chunks > 1:
            d_idx[1] = idx_start(1)
        d_idx[0].wait()
        d_g[0] = gather_start(0)
        for i in range(n_chunks):
            d_g[i].wait()
            d_s[i] = store_start(i)
            if i + 1 < n_chunks:
                if i - 1 >= 0:
                    for d in d_s[i - 1]:
                        d.wait()
                d_idx[i + 1].wait()
                d_g[i + 1] = gather_start(i + 1)
            if i + 2 < n_chunks:
                d_idx[i + 2] = idx_start(i + 2)
        if n_chunks >= 2:
            for d in d_s[n_chunks - 2]:
                d.wait()
        for d in d_s[n_chunks - 1]:
            d.wait()

    return gather_kernel


def kernel(x, embeddings):
    nb, s = x.shape
    V, d = embeddings.shape
    xf = x.reshape(nb * s).astype(jnp.int32)
    return _build(nb, s, V)(xf, embeddings)


# speedup vs baseline: 1.7524x; 1.0094x over previous
import functools

import jax
import jax.numpy as jnp
from jax import lax
from jax.experimental import pallas as pl
from jax.experimental.pallas import tpu as pltpu
from jax.experimental.pallas import tpu_sc as plsc

EMBEDDING_DIM = 32
CHUNK = 1664


@functools.cache
def _build(B, D):
    info = plsc.get_sparse_core_info()
    NC, NS = info.num_cores, info.num_subcores
    NW = NC * NS
    assert B % NW == 0
    b_per_w = B // NW
    assert b_per_w % CHUNK == 0
    n_chunks = b_per_w // CHUNK

    mesh = plsc.VectorSubcoreMesh(core_axis_name="c", subcore_axis_name="s")

    @functools.partial(
        pl.kernel,
        mesh=mesh,
        out_type=jax.ShapeDtypeStruct((B, D), jnp.float32),
        scratch_types=[
            pltpu.VMEM((CHUNK,), jnp.int32),
            pltpu.VMEM((CHUNK,), jnp.int32),
            pltpu.VMEM((CHUNK, D), jnp.float32),
            pltpu.VMEM((CHUNK, D), jnp.float32),
            pltpu.SemaphoreType.DMA,
            pltpu.SemaphoreType.DMA,
            pltpu.SemaphoreType.DMA,
            pltpu.SemaphoreType.DMA,
            pltpu.SemaphoreType.DMA,
            pltpu.SemaphoreType.DMA,
        ],
        compiler_params=pltpu.CompilerParams(use_tc_tiling_on_sc=False),
    )
    def gather_kernel(idx_hbm, table_hbm, out_hbm,
                      i0, i1, r0, r1, si0, si1, sg0, sg1, ss0, ss1):
        idx_bufs, rows_bufs = [i0, i1], [r0, r1]
        sem_i, sem_g, sem_s = [si0, si1], [sg0, sg1], [ss0, ss1]
        wid = lax.axis_index("s") * NC + lax.axis_index("c")
        base = wid * b_per_w

        def idx_start(i):
            off = base + i * CHUNK
            return pltpu.async_copy(
                idx_hbm.at[pl.ds(off, CHUNK)], idx_bufs[i % 2], sem_i[i % 2])

        def gather_start(i):
            return pltpu.async_copy(
                table_hbm.at[idx_bufs[i % 2]], rows_bufs[i % 2], sem_g[i % 2])

        def store_start(i):
            off = base + i * CHUNK
            return pltpu.async_copy(
                rows_bufs[i % 2], out_hbm.at[pl.ds(off, CHUNK)], sem_s[i % 2])

        d_idx, d_g, d_s = {}, {}, {}
        d_idx[0] = idx_start(0)
        if n_chunks > 1:
            d_idx[1] = idx_start(1)
        d_idx[0].wait()
        d_g[0] = gather_start(0)
        for i in range(n_chunks):
            d_g[i].wait()
            d_s[i] = store_start(i)
            if i + 1 < n_chunks:
                if i - 1 >= 0:
                    d_s[i - 1].wait()
                d_idx[i + 1].wait()
                d_g[i + 1] = gather_start(i + 1)
            if i + 2 < n_chunks:
                d_idx[i + 2] = idx_start(i + 2)
        if n_chunks >= 2:
            d_s[n_chunks - 2].wait()
        d_s[n_chunks - 1].wait()

    return gather_kernel


def kernel(x, embeddings):
    n, s = x.shape
    B = n * s
    xf = x.reshape(B).astype(jnp.int32)
    out = _build(B, EMBEDDING_DIM)(xf, embeddings)
    return out.reshape(n, s, EMBEDDING_DIM)
